# Initial kernel scaffold; baseline (speedup 1.0000x reference)
import jax
import jax.numpy as jnp
from jax.experimental import pallas as pl


def kernel(interactions, users_features, items_features, user_emb, item_emb, user_bias, item_bias, uf_tables, if_tables, Wu, bu, Wi, bi):
    raise NotImplementedError("write your pallas kernel here")


# speedup vs baseline: 1.0225x; 1.0225x over previous
import functools

import jax
import jax.numpy as jnp
from jax import lax
from jax.experimental import pallas as pl
from jax.experimental.pallas import tpu as pltpu
from jax.experimental.pallas import tpu_sc as plsc

B = 16384
NU = 1000000
D = 64
F = 8
NF = 4
CARD = 10000
H = D + NF * F

NC = 2
NS = 16
NW = NC * NS
BPW = B // NW
FPW = BPW * NF
IC = 128
RPW = BPW // IC
FRPW = FPW // IC

_mesh = plsc.VectorSubcoreMesh(core_axis_name="c", subcore_axis_name="s")


@functools.partial(
    pl.kernel,
    mesh=_mesh,
    out_type=[
        jax.ShapeDtypeStruct((B, D), jnp.float32),
        jax.ShapeDtypeStruct((B, D), jnp.float32),
        jax.ShapeDtypeStruct((B * NF, F), jnp.float32),
        jax.ShapeDtypeStruct((B * NF, F), jnp.float32),
        jax.ShapeDtypeStruct((B,), jnp.float32),
        jax.ShapeDtypeStruct((B,), jnp.float32),
    ],
    scratch_types=[
        pltpu.VMEM((RPW, IC), jnp.int32),
        pltpu.VMEM((RPW, IC), jnp.int32),
        pltpu.VMEM((FRPW, IC), jnp.int32),
        pltpu.VMEM((FRPW, IC), jnp.int32),
        pltpu.VMEM((RPW, IC), jnp.int32),
        pltpu.VMEM((RPW, IC), jnp.int32),
        pltpu.VMEM((BPW,), jnp.int32),
        pltpu.VMEM((BPW,), jnp.int32),
        pltpu.VMEM((BPW, D), jnp.float32),
        pltpu.VMEM((BPW, D), jnp.float32),
        pltpu.VMEM((FPW, F), jnp.float32),
        pltpu.VMEM((FPW, F), jnp.float32),
        pltpu.VMEM((BPW, 16), jnp.float32),
        pltpu.VMEM((BPW, 16), jnp.float32),
        pltpu.VMEM((BPW,), jnp.float32),
        pltpu.VMEM((BPW,), jnp.float32),
        pltpu.SemaphoreType.DMA,
    ],
    compiler_params=pltpu.CompilerParams(use_tc_tiling_on_sc=False,
                                         needs_layout_passes=False),
)
def _sc_gather(uemb, iemb, ub16, ib16, uft, ift,
               uidx_h, iidx_h, ufidx_h, ifidx_h,
               ubrow_h, ibrow_h, ulane_h, ilane_h,
               u_out, i_out, uf_out, if_out, ub_out, ib_out,
               uidx_v, iidx_v, ufidx_v, ifidx_v,
               ubrow_v, ibrow_v, ulane_v, ilane_v,
               urows_v, irows_v, ufr_v, ifr_v,
               ubr16_v, ibr16_v, ubv, ibv, sem):
    wid = lax.axis_index("s") * NC + lax.axis_index("c")
    base = wid * BPW
    fbase = wid * FPW
    pltpu.sync_copy(uidx_h.at[pl.ds(wid * RPW, RPW)], uidx_v)
    pltpu.sync_copy(iidx_h.at[pl.ds(wid * RPW, RPW)], iidx_v)
    pltpu.sync_copy(ufidx_h.at[pl.ds(wid * FRPW, FRPW)], ufidx_v)
    pltpu.sync_copy(ifidx_h.at[pl.ds(wid * FRPW, FRPW)], ifidx_v)
    pltpu.sync_copy(ubrow_h.at[pl.ds(wid * RPW, RPW)], ubrow_v)
    pltpu.sync_copy(ibrow_h.at[pl.ds(wid * RPW, RPW)], ibrow_v)
    pltpu.sync_copy(ulane_h.at[pl.ds(base, BPW)], ulane_v)
    pltpu.sync_copy(ilane_h.at[pl.ds(base, BPW)], ilane_v)
    copies = []
    for j in range(RPW):
        dst = pl.ds(j * IC, IC)
        copies.append(pltpu.async_copy(uemb.at[uidx_v.at[j]], urows_v.at[dst], sem))
        copies.append(pltpu.async_copy(iemb.at[iidx_v.at[j]], irows_v.at[dst], sem))
        copies.append(pltpu.async_copy(ub16.at[ubrow_v.at[j]], ubr16_v.at[dst], sem))
        copies.append(pltpu.async_copy(ib16.at[ibrow_v.at[j]], ibr16_v.at[dst], sem))
    for j in range(FRPW):
        dst = pl.ds(j * IC, IC)
        copies.append(pltpu.async_copy(uft.at[ufidx_v.at[j]], ufr_v.at[dst], sem))
        copies.append(pltpu.async_copy(ift.at[ifidx_v.at[j]], ifr_v.at[dst], sem))
    for c in copies:
        c.wait()
    for k in range(BPW // 16):
        rows = lax.iota(jnp.int32, 16) + k * 16
        ubv[pl.ds(k * 16, 16)] = plsc.load_gather(
            ubr16_v, [rows, ulane_v[pl.ds(k * 16, 16)]])
        ibv[pl.ds(k * 16, 16)] = plsc.load_gather(
            ibr16_v, [rows, ilane_v[pl.ds(k * 16, 16)]])
    pltpu.sync_copy(urows_v, u_out.at[pl.ds(base, BPW)])
    pltpu.sync_copy(irows_v, i_out.at[pl.ds(base, BPW)])
    pltpu.sync_copy(ufr_v, uf_out.at[pl.ds(fbase, FPW)])
    pltpu.sync_copy(ifr_v, if_out.at[pl.ds(fbase, FPW)])
    pltpu.sync_copy(ubv, ub_out.at[pl.ds(base, BPW)])
    pltpu.sync_copy(ibv, ib_out.at[pl.ds(base, BPW)])


BLK = 2048
_PREC = lax.Precision.HIGHEST


def _dense_body(ur, uf, ir, if_r, ub, ib, wue, wuf, bu_r, wie, wif, bi_r, out):
    ufact = (jnp.dot(ur[...], wue[...], preferred_element_type=jnp.float32,
                     precision=_PREC)
             + jnp.dot(uf[...], wuf[...], preferred_element_type=jnp.float32,
                       precision=_PREC)
             + bu_r[...])
    ifact = (jnp.dot(ir[...], wie[...], preferred_element_type=jnp.float32,
                     precision=_PREC)
             + jnp.dot(if_r[...], wif[...], preferred_element_type=jnp.float32,
                       precision=_PREC)
             + bi_r[...])
    out[...] = jnp.sum(ufact * ifact, axis=1, keepdims=True) + ub[...] + ib[...]


def _dense(u_rows, ufeat, i_rows, ifeat, ub, ib, wue, wuf, bu2, wie, wif, bi2):
    fw = NF * F
    return pl.pallas_call(
        _dense_body,
        grid=(B // BLK,),
        in_specs=[
            pl.BlockSpec((BLK, D), lambda i: (i, 0)),
            pl.BlockSpec((BLK, fw), lambda i: (i, 0)),
            pl.BlockSpec((BLK, D), lambda i: (i, 0)),
            pl.BlockSpec((BLK, fw), lambda i: (i, 0)),
            pl.BlockSpec((BLK, 1), lambda i: (i, 0)),
            pl.BlockSpec((BLK, 1), lambda i: (i, 0)),
            pl.BlockSpec((D, H), lambda i: (0, 0)),
            pl.BlockSpec((fw, H), lambda i: (0, 0)),
            pl.BlockSpec((1, H), lambda i: (0, 0)),
            pl.BlockSpec((D, H), lambda i: (0, 0)),
            pl.BlockSpec((fw, H), lambda i: (0, 0)),
            pl.BlockSpec((1, H), lambda i: (0, 0)),
        ],
        out_specs=pl.BlockSpec((BLK, 1), lambda i: (i, 0)),
        out_shape=jax.ShapeDtypeStruct((B, 1), jnp.float32),
    )(u_rows, ufeat, i_rows, ifeat, ub, ib, wue, wuf, bu2, wie, wif, bi2)


def kernel(interactions, users_features, items_features, user_emb, item_emb,
           user_bias, item_bias, uf_tables, if_tables, Wu, bu, Wi, bi):
    uidx = interactions[:, 0].astype(jnp.int32)
    iidx = interactions[:, 1].astype(jnp.int32)
    foff = (jnp.arange(NF, dtype=jnp.int32) * CARD)[None, :]
    ufidx = (users_features.astype(jnp.int32) + foff).reshape(B * NF // IC, IC)
    ifidx = (items_features.astype(jnp.int32) + foff).reshape(B * NF // IC, IC)
    ubrow = (uidx >> 4).reshape(B // IC, IC)
    ibrow = (iidx >> 4).reshape(B // IC, IC)
    ulane = uidx & 15
    ilane = iidx & 15
    nbr = (NU + 16) // 16 * 16
    ub16 = jnp.pad(user_bias.reshape(-1), (0, nbr - (NU + 1))).reshape(-1, 16)
    ib16 = jnp.pad(item_bias.reshape(-1), (0, nbr - (NU + 1))).reshape(-1, 16)
    uidx = uidx.reshape(B // IC, IC)
    iidx = iidx.reshape(B // IC, IC)
    uft = uf_tables.reshape(NF * CARD, F)
    ift = if_tables.reshape(NF * CARD, F)

    u_rows, i_rows, uf_rows, if_rows, ub, ib = _sc_gather(
        user_emb, item_emb, ub16, ib16, uft, ift,
        uidx, iidx, ufidx, ifidx, ubrow, ibrow, ulane, ilane)

    ufeat = uf_rows.reshape(B, NF * F)
    ifeat = if_rows.reshape(B, NF * F)
    yh = _dense(u_rows, ufeat, i_rows, ifeat,
                ub.reshape(B, 1), ib.reshape(B, 1),
                Wu[:, :D].T, Wu[:, D:].T, bu.reshape(1, H),
                Wi[:, :D].T, Wi[:, D:].T, bi.reshape(1, H))
    return yh.reshape(B)
